# baseline (speedup 1.0000x reference)
import functools

import jax
import jax.numpy as jnp
from jax import lax
from jax.experimental import pallas as pl
from jax.experimental.pallas import tpu as pltpu
from jax.experimental.pallas import tpu_sc as plsc

L = 16
P = 48


def _make_sc_resample(HW_in, HW_out, C, H, W):
    info = plsc.get_sparse_core_info()
    NW = info.num_cores * info.num_subcores
    per_w = HW_out // NW
    n_chunks = per_w // P

    mesh = plsc.VectorSubcoreMesh(core_axis_name="c", subcore_axis_name="s")

    @functools.partial(
        pl.kernel,
        mesh=mesh,
        out_type=jax.ShapeDtypeStruct((HW_out, C), jnp.float32),
        scratch_types=[
            pltpu.VMEM((per_w,), jnp.float32),
            pltpu.VMEM((per_w,), jnp.float32),
            pltpu.VMEM((2, 4, P), jnp.int32),
            pltpu.VMEM((2, 4, P), jnp.float32),
            pltpu.VMEM((2, 2, P, C), jnp.float32),
            pltpu.VMEM((2, P, C), jnp.float32),
            pltpu.SemaphoreType.DMA,
            pltpu.SemaphoreType.DMA,
            pltpu.SemaphoreType.DMA,
        ],
    )
    def body(xt_hbm, xs_hbm, ys_hbm, out_hbm,
             xsv, ysv, idxv, wv, rows, outv, g0, g1, osem):
        cid = lax.axis_index("c")
        sid = lax.axis_index("s")
        wid = sid * info.num_cores + cid
        base_w = wid * per_w

        pltpu.sync_copy(xs_hbm.at[pl.ds(base_w, per_w)], xsv)
        pltpu.sync_copy(ys_hbm.at[pl.ds(base_w, per_w)], ysv)

        def gsem(slot):
            return [g0, g1][slot]

        def prep(c, cp):
            for g in range(P // L):
                sl = pl.ds(c * P + g * L, L)
                dsl = pl.ds(g * L, L)
                xs16 = xsv[sl]
                ys16 = ysv[sl]
                x0 = xs16.astype(jnp.int32)
                y0 = ys16.astype(jnp.int32)
                dx = xs16 - x0.astype(jnp.float32)
                dy = ys16 - y0.astype(jnp.float32)
                x1 = jnp.minimum(x0 + 1, W - 1)
                y1 = jnp.minimum(y0 + 1, H - 1)
                r0 = y0 * W
                r1 = y1 * W
                idxv[cp, 0, dsl] = r0 + x0
                idxv[cp, 1, dsl] = r0 + x1
                idxv[cp, 2, dsl] = r1 + x0
                idxv[cp, 3, dsl] = r1 + x1
                wv[cp, 0, dsl] = (1.0 - dx) * (1.0 - dy)
                wv[cp, 1, dsl] = dx * (1.0 - dy)
                wv[cp, 2, dsl] = (1.0 - dx) * dy
                wv[cp, 3, dsl] = dx * dy

        def fire(pair, slot, cp):
            for h in range(2):
                pltpu.async_copy(
                    xt_hbm.at[idxv.at[cp, 2 * pair + h]],
                    rows.at[slot, h], gsem(slot))

        def wait_pair(pair, slot, cp):
            for h in range(2):
                pltpu.make_async_copy(
                    xt_hbm.at[idxv.at[cp, 2 * pair + h]],
                    rows.at[slot, h], gsem(slot)).wait()

        def blend(pair, slot, cp, oc):
            first = pair == 0

            def grp(g, carry):
                wa16 = wv[cp, 2 * pair, pl.ds(g * L, L)]
                wb16 = wv[cp, 2 * pair + 1, pl.ds(g * L, L)]
                for pp in range(L):
                    wa = wa16[pp]
                    wb = wb16[pp]
                    p = g * L + pp
                    for j in range(C // L):
                        slj = pl.ds(j * L, L)
                        v = (rows[slot, 0, p, slj] * wa
                             + rows[slot, 1, p, slj] * wb)
                        if first:
                            outv[oc, p, slj] = v
                        else:
                            plsc.addupdate(outv.at[oc, p, slj], v)
                return carry

            lax.fori_loop(0, P // L, grp, 0)

        def issue_out(c, oc):
            pltpu.async_copy(
                outv.at[oc], out_hbm.at[pl.ds(base_w + c * P, P)], osem)

        def drain_out(c, oc):
            pltpu.make_async_copy(
                outv.at[oc], out_hbm.at[pl.ds(base_w + c * P, P)],
                osem).wait()

        prep(0, 0)
        fire(0, 0, 0)

        def chunk_step(c, carry):
            cp = lax.rem(c, 2)
            cpn = 1 - cp
            oc = cp
            fire(1, 1, cp)

            @pl.when(c + 1 < n_chunks)
            def _():
                prep(c + 1, cpn)

            wait_pair(0, 0, cp)

            @pl.when(c >= 2)
            def _():
                drain_out(c - 2, oc)

            blend(0, 0, cp, oc)

            @pl.when(c + 1 < n_chunks)
            def _():
                fire(0, 0, cpn)

            wait_pair(1, 1, cp)
            blend(1, 1, cp, oc)
            issue_out(c, oc)
            return carry

        lax.fori_loop(0, n_chunks, chunk_step, 0)
        drain_out(n_chunks - 2, 0)
        drain_out(n_chunks - 1, 1)

    return body


def kernel(x, sample_map):
    B, C, H, W = x.shape
    Ho, Wo = sample_map.shape[0], sample_map.shape[1]
    HW_in = H * W
    HW_out = Ho * Wo
    xt = x.reshape(C, HW_in).T
    xs = sample_map[..., 0].reshape(HW_out)
    ys = sample_map[..., 1].reshape(HW_out)
    sc = _make_sc_resample(HW_in, HW_out, C, H, W)
    out_t = sc(xt, xs, ys)
    return out_t.T.reshape(B, C, Ho, Wo)


# speedup vs baseline: 1.1806x; 1.1806x over previous
import functools

import jax
import jax.numpy as jnp
from jax import lax
from jax.experimental import pallas as pl
from jax.experimental.pallas import tpu as pltpu
from jax.experimental.pallas import tpu_sc as plsc

L = 16
P = 64


def _make_sc_resample(HW_in, HW_out, C, H, W):
    info = plsc.get_sparse_core_info()
    NW = info.num_cores * info.num_subcores
    per_w = HW_out // NW
    n_chunks = per_w // P

    mesh = plsc.VectorSubcoreMesh(core_axis_name="c", subcore_axis_name="s")

    @functools.partial(
        pl.kernel,
        mesh=mesh,
        out_type=jax.ShapeDtypeStruct((HW_out, C), jnp.float32),
        scratch_types=[
            pltpu.VMEM((per_w,), jnp.float32),
            pltpu.VMEM((per_w,), jnp.float32),
            pltpu.VMEM((2, 4, P), jnp.int32),
            pltpu.VMEM((2, 4, P), jnp.float32),
            pltpu.VMEM((2, P, C), jnp.float32),
            pltpu.VMEM((2, P, C), jnp.float32),
            pltpu.SemaphoreType.DMA,
            pltpu.SemaphoreType.DMA,
            pltpu.SemaphoreType.DMA,
        ],
    )
    def body(xt_hbm, xs_hbm, ys_hbm, out_hbm,
             xsv, ysv, idxv, wv, rows, outv, g0, g1, osem):
        cid = lax.axis_index("c")
        sid = lax.axis_index("s")
        wid = sid * info.num_cores + cid
        base_w = wid * per_w

        pltpu.sync_copy(xs_hbm.at[pl.ds(base_w, per_w)], xsv)
        pltpu.sync_copy(ys_hbm.at[pl.ds(base_w, per_w)], ysv)

        def gsem(slot):
            return [g0, g1][slot]

        def prep(c, cp):
            for g in range(P // L):
                sl = pl.ds(c * P + g * L, L)
                dsl = pl.ds(g * L, L)
                xs16 = xsv[sl]
                ys16 = ysv[sl]
                x0 = xs16.astype(jnp.int32)
                y0 = ys16.astype(jnp.int32)
                dx = xs16 - x0.astype(jnp.float32)
                dy = ys16 - y0.astype(jnp.float32)
                x1 = jnp.minimum(x0 + 1, W - 1)
                y1 = jnp.minimum(y0 + 1, H - 1)
                r0 = y0 * W
                r1 = y1 * W
                idxv[cp, 0, dsl] = r0 + x0
                idxv[cp, 1, dsl] = r0 + x1
                idxv[cp, 2, dsl] = r1 + x0
                idxv[cp, 3, dsl] = r1 + x1
                wv[cp, 0, dsl] = (1.0 - dx) * (1.0 - dy)
                wv[cp, 1, dsl] = dx * (1.0 - dy)
                wv[cp, 2, dsl] = (1.0 - dx) * dy
                wv[cp, 3, dsl] = dx * dy

        def fire(k, slot, cp):
            pltpu.async_copy(
                xt_hbm.at[idxv.at[cp, k]], rows.at[slot], gsem(slot))

        def wait_gather(k, slot, cp):
            pltpu.make_async_copy(
                xt_hbm.at[idxv.at[cp, k]], rows.at[slot], gsem(slot)).wait()

        def blend(k, slot, cp, oc):
            def grp(g, carry):
                wk16 = wv[cp, k, pl.ds(g * L, L)]
                for pp in range(L):
                    wk = wk16[pp]
                    p = g * L + pp
                    for j in range(C // L):
                        slj = pl.ds(j * L, L)
                        v = rows[slot, p, slj] * wk
                        if k == 0:
                            outv[oc, p, slj] = v
                        else:
                            plsc.addupdate(outv.at[oc, p, slj], v)
                return carry

            lax.fori_loop(0, P // L, grp, 0)

        def issue_out(c, oc):
            pltpu.async_copy(
                outv.at[oc], out_hbm.at[pl.ds(base_w + c * P, P)], osem)

        def drain_out(c, oc):
            pltpu.make_async_copy(
                outv.at[oc], out_hbm.at[pl.ds(base_w + c * P, P)],
                osem).wait()

        prep(0, 0)
        fire(0, 0, 0)

        def chunk_step(c, carry):
            cp = lax.rem(c, 2)
            cpn = 1 - cp
            oc = cp
            for k in range(4):
                nslot = k % 2
                if k < 3:
                    fire(k + 1, (k + 1) % 2, cp)
                else:
                    @pl.when(c + 1 < n_chunks)
                    def _():
                        fire(0, 0, cpn)
                if k == 0:
                    @pl.when(c + 1 < n_chunks)
                    def _():
                        prep(c + 1, cpn)
                wait_gather(k, nslot, cp)
                if k == 0:
                    @pl.when(c >= 2)
                    def _():
                        drain_out(c - 2, oc)
                blend(k, nslot, cp, oc)
                if k == 3:
                    issue_out(c, oc)
            return carry

        lax.fori_loop(0, n_chunks, chunk_step, 0)
        drain_out(n_chunks - 2, 0)
        drain_out(n_chunks - 1, 1)

    return body


def kernel(x, sample_map):
    B, C, H, W = x.shape
    Ho, Wo = sample_map.shape[0], sample_map.shape[1]
    HW_in = H * W
    HW_out = Ho * Wo
    xt = x.reshape(C, HW_in).T
    xs = sample_map[..., 0].reshape(HW_out)
    ys = sample_map[..., 1].reshape(HW_out)
    sc = _make_sc_resample(HW_in, HW_out, C, H, W)
    out_t = sc(xt, xs, ys)
    return out_t.T.reshape(B, C, Ho, Wo)


# speedup vs baseline: 2.7762x; 2.3515x over previous
import functools

import jax
import jax.numpy as jnp
from jax import lax
from jax.experimental import pallas as pl
from jax.experimental.pallas import tpu as pltpu
from jax.experimental.pallas import tpu_sc as plsc

L = 16
P = 64


def _make_sc_resample(HW_in, HW_out, C, H, W):
    info = plsc.get_sparse_core_info()
    NW = info.num_cores * info.num_subcores
    per_w = HW_out // NW
    n_chunks = per_w // P

    mesh = plsc.VectorSubcoreMesh(core_axis_name="c", subcore_axis_name="s")

    @functools.partial(
        pl.kernel,
        mesh=mesh,
        out_type=jax.ShapeDtypeStruct((HW_out, C), jnp.float32),
        scratch_types=[
            pltpu.VMEM((per_w,), jnp.float32),
            pltpu.VMEM((per_w,), jnp.float32),
            pltpu.VMEM((2, 4, P), jnp.int32),
            pltpu.VMEM((2, 4, P), jnp.float32),
            pltpu.VMEM((2, P, C), jnp.float32),
            pltpu.VMEM((P, C), jnp.float32),
            pltpu.SemaphoreType.DMA,
            pltpu.SemaphoreType.DMA,
            pltpu.SemaphoreType.DMA,
        ],
    )
    def body(xt_hbm, xs_hbm, ys_hbm, out_hbm,
             xsv, ysv, idxv, wv, rows, outv, g0, g1, osem):
        cid = lax.axis_index("c")
        sid = lax.axis_index("s")
        wid = sid * info.num_cores + cid
        base_w = wid * per_w

        pltpu.sync_copy(xs_hbm.at[pl.ds(base_w, per_w)], xsv)
        pltpu.sync_copy(ys_hbm.at[pl.ds(base_w, per_w)], ysv)

        def gsem(slot):
            return [g0, g1][slot]

        def prep(c, cp):
            for g in range(P // L):
                sl = pl.ds(c * P + g * L, L)
                dsl = pl.ds(g * L, L)
                xs16 = xsv[sl]
                ys16 = ysv[sl]
                x0 = xs16.astype(jnp.int32)
                y0 = ys16.astype(jnp.int32)
                dx = xs16 - x0.astype(jnp.float32)
                dy = ys16 - y0.astype(jnp.float32)
                x1 = jnp.minimum(x0 + 1, W - 1)
                y1 = jnp.minimum(y0 + 1, H - 1)
                r0 = y0 * W
                r1 = y1 * W
                idxv[cp, 0, dsl] = r0 + x0
                idxv[cp, 1, dsl] = r0 + x1
                idxv[cp, 2, dsl] = r1 + x0
                idxv[cp, 3, dsl] = r1 + x1
                wv[cp, 0, dsl] = (1.0 - dx) * (1.0 - dy)
                wv[cp, 1, dsl] = dx * (1.0 - dy)
                wv[cp, 2, dsl] = (1.0 - dx) * dy
                wv[cp, 3, dsl] = dx * dy

        def fire(k, slot, cp):
            pltpu.async_copy(
                xt_hbm.at[idxv.at[cp, k]], rows.at[slot], gsem(slot))

        def wait_gather(k, slot, cp):
            pltpu.make_async_copy(
                xt_hbm.at[idxv.at[cp, k]], rows.at[slot], gsem(slot)).wait()

        def blend(k, slot, cp):
            def grp(g, carry):
                wk16 = wv[cp, k, pl.ds(g * L, L)]
                for pp in range(L):
                    wk = wk16[pp]
                    p = g * L + pp
                    for j in range(C // L):
                        slj = pl.ds(j * L, L)
                        v = rows[slot, p, slj] * wk
                        if k == 0:
                            outv[p, slj] = v
                        else:
                            plsc.addupdate(outv.at[p, slj], v)
                return carry

            lax.fori_loop(0, P // L, grp, 0)

        def issue_out(c):
            pltpu.async_copy(
                outv, out_hbm.at[pl.ds(base_w + c * P, P)], osem)

        def drain_out(c):
            pltpu.make_async_copy(
                outv, out_hbm.at[pl.ds(base_w + c * P, P)], osem).wait()

        prep(0, 0)
        fire(0, 0, 0)

        def chunk_step(c, carry):
            cp = lax.rem(c, 2)
            cpn = 1 - cp
            for k in range(4):
                nslot = k % 2
                if k < 3:
                    fire(k + 1, (k + 1) % 2, cp)
                else:
                    @pl.when(c + 1 < n_chunks)
                    def _():
                        fire(0, 0, cpn)
                if k == 0:
                    @pl.when(c + 1 < n_chunks)
                    def _():
                        prep(c + 1, cpn)
                wait_gather(k, nslot, cp)
                if k == 0:
                    @pl.when(c >= 1)
                    def _():
                        drain_out(c - 1)
                blend(k, nslot, cp)
                if k == 3:
                    issue_out(c)
            return carry

        lax.fori_loop(0, n_chunks, chunk_step, 0)
        drain_out(n_chunks - 1)

    return body


def kernel(x, sample_map):
    B, C, H, W = x.shape
    Ho, Wo = sample_map.shape[0], sample_map.shape[1]
    HW_in = H * W
    HW_out = Ho * Wo
    xt = x.reshape(C, HW_in).T
    xs = sample_map[..., 0].reshape(HW_out)
    ys = sample_map[..., 1].reshape(HW_out)
    sc = _make_sc_resample(HW_in, HW_out, C, H, W)
    out_t = sc(xt, xs, ys)
    return out_t.T.reshape(B, C, Ho, Wo)


# speedup vs baseline: 3.0211x; 1.0882x over previous
import functools

import jax
import jax.numpy as jnp
from jax import lax
from jax.experimental import pallas as pl
from jax.experimental.pallas import tpu as pltpu
from jax.experimental.pallas import tpu_sc as plsc

L = 16
P = 96


def _make_sc_resample(HW_in, HW_out, C, H, W):
    info = plsc.get_sparse_core_info()
    NW = info.num_cores * info.num_subcores
    per_w = HW_out // NW
    n_chunks = per_w // P

    mesh = plsc.VectorSubcoreMesh(core_axis_name="c", subcore_axis_name="s")

    @functools.partial(
        pl.kernel,
        mesh=mesh,
        out_type=jax.ShapeDtypeStruct((HW_out, C), jnp.float32),
        scratch_types=[
            pltpu.VMEM((per_w,), jnp.float32),
            pltpu.VMEM((per_w,), jnp.float32),
            pltpu.VMEM((2, 4, P), jnp.int32),
            pltpu.VMEM((2, 4, P), jnp.float32),
            pltpu.VMEM((2, P, C), jnp.float32),
            pltpu.VMEM((P, C), jnp.float32),
            pltpu.SemaphoreType.DMA,
            pltpu.SemaphoreType.DMA,
            pltpu.SemaphoreType.DMA,
        ],
    )
    def body(xt_hbm, xs_hbm, ys_hbm, out_hbm,
             xsv, ysv, idxv, wv, rows, outv, g0, g1, osem):
        cid = lax.axis_index("c")
        sid = lax.axis_index("s")
        wid = sid * info.num_cores + cid
        base_w = wid * per_w

        pltpu.sync_copy(xs_hbm.at[pl.ds(base_w, per_w)], xsv)
        pltpu.sync_copy(ys_hbm.at[pl.ds(base_w, per_w)], ysv)

        def gsem(slot):
            return [g0, g1][slot]

        def prep(c, cp):
            for g in range(P // L):
                sl = pl.ds(c * P + g * L, L)
                dsl = pl.ds(g * L, L)
                xs16 = xsv[sl]
                ys16 = ysv[sl]
                x0 = xs16.astype(jnp.int32)
                y0 = ys16.astype(jnp.int32)
                dx = xs16 - x0.astype(jnp.float32)
                dy = ys16 - y0.astype(jnp.float32)
                x1 = jnp.minimum(x0 + 1, W - 1)
                y1 = jnp.minimum(y0 + 1, H - 1)
                r0 = y0 * W
                r1 = y1 * W
                idxv[cp, 0, dsl] = r0 + x0
                idxv[cp, 1, dsl] = r0 + x1
                idxv[cp, 2, dsl] = r1 + x0
                idxv[cp, 3, dsl] = r1 + x1
                wv[cp, 0, dsl] = (1.0 - dx) * (1.0 - dy)
                wv[cp, 1, dsl] = dx * (1.0 - dy)
                wv[cp, 2, dsl] = (1.0 - dx) * dy
                wv[cp, 3, dsl] = dx * dy

        H2 = P // 2

        def fire(k, slot, cp):
            for h in range(2):
                pltpu.async_copy(
                    xt_hbm.at[idxv.at[cp, k, pl.ds(h * H2, H2)]],
                    rows.at[slot, pl.ds(h * H2, H2)], gsem(slot))

        def wait_gather(k, slot, cp):
            for h in range(2):
                pltpu.make_async_copy(
                    xt_hbm.at[idxv.at[cp, k, pl.ds(h * H2, H2)]],
                    rows.at[slot, pl.ds(h * H2, H2)], gsem(slot)).wait()

        def blend(k, slot, cp):
            def grp(g, carry):
                wk16 = wv[cp, k, pl.ds(g * L, L)]
                for pp in range(L):
                    wk = wk16[pp]
                    p = g * L + pp
                    for j in range(C // L):
                        slj = pl.ds(j * L, L)
                        v = rows[slot, p, slj] * wk
                        if k == 0:
                            outv[p, slj] = v
                        else:
                            plsc.addupdate(outv.at[p, slj], v)
                return carry

            lax.fori_loop(0, P // L, grp, 0)

        def issue_out(c):
            pltpu.async_copy(
                outv, out_hbm.at[pl.ds(base_w + c * P, P)], osem)

        def drain_out(c):
            pltpu.make_async_copy(
                outv, out_hbm.at[pl.ds(base_w + c * P, P)], osem).wait()

        prep(0, 0)
        fire(0, 0, 0)

        def chunk_step(c, carry):
            cp = lax.rem(c, 2)
            cpn = 1 - cp
            for k in range(4):
                nslot = k % 2
                if k < 3:
                    fire(k + 1, (k + 1) % 2, cp)
                else:
                    @pl.when(c + 1 < n_chunks)
                    def _():
                        fire(0, 0, cpn)
                if k == 0:
                    @pl.when(c + 1 < n_chunks)
                    def _():
                        prep(c + 1, cpn)
                wait_gather(k, nslot, cp)
                if k == 0:
                    @pl.when(c >= 1)
                    def _():
                        drain_out(c - 1)
                blend(k, nslot, cp)
                if k == 3:
                    issue_out(c)
            return carry

        lax.fori_loop(0, n_chunks, chunk_step, 0)
        drain_out(n_chunks - 1)

    return body


def kernel(x, sample_map):
    B, C, H, W = x.shape
    Ho, Wo = sample_map.shape[0], sample_map.shape[1]
    HW_in = H * W
    HW_out = Ho * Wo
    xt = x.reshape(C, HW_in).T
    xs = sample_map[..., 0].reshape(HW_out)
    ys = sample_map[..., 1].reshape(HW_out)
    sc = _make_sc_resample(HW_in, HW_out, C, H, W)
    out_t = sc(xt, xs, ys)
    return out_t.T.reshape(B, C, Ho, Wo)
